# baseline (speedup 1.0000x reference)
import functools

import jax
import jax.numpy as jnp
from jax import lax
from jax.experimental import pallas as pl
from jax.experimental.pallas import tpu as pltpu
from jax.experimental.pallas import tpu_sc as plsc

HID = 128
N = 49152
E = 786432
N_GI = 2048
B = 1024
NPW = 48
ISL_PER_G = 24
NCELL = 64
QW = 32
NQ = 4
NC = 2
NS = 16
CHUNK = 128
SUP = 8
ROWS_PER_TILE = E // CHUNK // NS
N_SUP = ROWS_PER_TILE // SUP
STRIPE = N // NS

WCH = 128
DEGW = 16

@functools.cache
def _mesh():
    return plsc.VectorSubcoreMesh(core_axis_name="c", subcore_axis_name="s",
                                  num_cores=NC, num_subcores=NS)


def _fill2d16(ref, rows, width, value):
    v = jnp.full((16,), value, jnp.float32)
    nw = width // 16

    def body(k, _):
        ref[k // nw, pl.ds((k % nw) * 16, 16)] = v
        return 0

    lax.fori_loop(0, rows * nw, body, 0)


def _seg_sum_body(xflat, src2d, dst2d, agg, acc, srcv, dstv, gidx, rows,
                  wbuf, gs0, gs1, gs2, gs3, ss0, ss1, ss2, ss3, is0, is1):
    c = lax.axis_index("c")
    s = lax.axis_index("s")
    gsems = (gs0, gs1, gs2, gs3)
    ssems = (ss0, ss1, ss2, ss3)
    isems = (is0, is1)
    base0 = s * ROWS_PER_TILE

    _fill2d16(wbuf.at[0], WCH, QW, 0.0)

    for qq in range(2):
        q = c * 2 + qq

        def zbody(i, _):
            pltpu.sync_copy(wbuf.at[0],
                            acc.at[pl.ds(s * STRIPE + i * WCH, WCH), :])
            return 0

        lax.fori_loop(0, STRIPE // WCH, zbody, 0)
        plsc.subcore_barrier()

        def load_idx(sup, p, sync):
            rr = base0 + sup * SUP
            if sync:
                pltpu.sync_copy(src2d.at[pl.ds(rr, SUP)], srcv.at[p])
                pltpu.sync_copy(dst2d.at[pl.ds(rr, SUP)], dstv.at[p])
            else:
                pltpu.async_copy(src2d.at[pl.ds(rr, SUP)], srcv.at[p],
                                 isems[p])
                pltpu.async_copy(dst2d.at[pl.ds(rr, SUP)], dstv.at[p],
                                 isems[p])

        def wait_idx(p):
            pltpu.make_async_copy(src2d.at[pl.ds(0, SUP)], srcv.at[p],
                                  isems[p]).wait()
            pltpu.make_async_copy(dst2d.at[pl.ds(0, SUP)], dstv.at[p],
                                  isems[p]).wait()

        nv = CHUNK // 16

        def compute_gidx(p):
            def idx_body(k, _):
                j = k // nv
                o = (k % nv) * 16
                gidx[p, j, pl.ds(o, 16)] = srcv[p, j, pl.ds(o, 16)] + q * N
                return 0

            lax.fori_loop(0, SUP * nv, idx_body, 0)

        def fire_gather(p, j, b):
            return pltpu.async_copy(xflat.at[gidx.at[p, j]], rows.at[b],
                                    gsems[b])

        def fire_scatter(p, j, b):
            return pltpu.async_copy(rows.at[b], acc.at[dstv.at[p, j]],
                                    ssems[b], add=True)

        def process_sup(sup, p, prefetch):
            if prefetch:
                load_idx(sup + 1, p ^ 1, sync=False)
            ga = [fire_gather(p, j, j) for j in range(4)]
            sa = []
            for j in range(4):
                ga[j].wait()
                sa.append(fire_scatter(p, j, j))
            gb = []
            for j in range(4):
                sa[j].wait()
                gb.append(fire_gather(p, 4 + j, j))
            sb = []
            for j in range(4):
                gb[j].wait()
                sb.append(fire_scatter(p, 4 + j, j))
            for j in range(4):
                sb[j].wait()

        load_idx(0, 0, sync=True)
        compute_gidx(0)
        process_sup(0, 0, True)

        def pair_body(t, _):
            sup = 1 + 2 * t
            wait_idx(1)
            compute_gidx(1)
            process_sup(sup, 1, True)
            wait_idx(0)
            compute_gidx(0)
            process_sup(sup + 1, 0, True)
            return 0

        lax.fori_loop(0, (N_SUP - 2) // 2, pair_body, 0)

        wait_idx(1)
        compute_gidx(1)
        process_sup(N_SUP - 1, 1, False)
        plsc.subcore_barrier()

        def wbody(i, _):
            r = s * STRIPE + i * WCH
            pltpu.sync_copy(acc.at[pl.ds(r, WCH), :], wbuf.at[i % 2])
            pltpu.sync_copy(wbuf.at[i % 2], agg.at[q, pl.ds(r, WCH), :])
            return 0

        lax.fori_loop(0, STRIPE // WCH, wbody, 0)
        if qq == 0:
            _fill2d16(wbuf.at[0], WCH, QW, 0.0)
        plsc.subcore_barrier()


@functools.cache
def _seg_sum_kernel():
    return pl.kernel(
        _seg_sum_body,
        out_type=jax.ShapeDtypeStruct((NQ, N, QW), jnp.float32),
        mesh=_mesh(),
        compiler_params=pltpu.CompilerParams(use_tc_tiling_on_sc=False),
        scratch_types=[
            pltpu.VMEM_SHARED((N, QW), jnp.float32),
            pltpu.VMEM((2, SUP, CHUNK), jnp.int32),
            pltpu.VMEM((2, SUP, CHUNK), jnp.int32),
            pltpu.VMEM((2, SUP, CHUNK), jnp.int32),
            pltpu.VMEM((4, CHUNK, QW), jnp.float32),
            pltpu.VMEM((2, WCH, QW), jnp.float32),
        ] + [pltpu.SemaphoreType.DMA] * 10,
    )


def _seg_sum(xflat, src2d, dst2d):
    return _seg_sum_kernel()(xflat, src2d, dst2d)


def _deg_body(dst2d_a, dst2d_b, deg_a, deg_b, dacc, dstv, onesv, wbuf):
    c = lax.axis_index("c")
    s = lax.axis_index("s")

    _fill2d16(onesv, CHUNK, DEGW, 1.0)

    rows_per_tile = E // CHUNK // (NC * NS)

    for dst2d, deg in ((dst2d_a, deg_a), (dst2d_b, deg_b)):
        _fill2d16(wbuf.at[0], WCH, DEGW, 0.0)

        def zbody(i, _):
            pltpu.sync_copy(wbuf.at[0],
                            dacc.at[pl.ds(s * STRIPE + i * WCH, WCH), :])
            return 0

        lax.fori_loop(0, STRIPE // WCH, zbody, 0)
        plsc.subcore_barrier()

        def sup_body(t, _):
            base = (c * NS + s) * rows_per_tile + t * SUP
            pltpu.sync_copy(dst2d.at[pl.ds(base, SUP)], dstv)
            for j in range(SUP):
                pltpu.sync_copy(onesv, dacc.at[dstv.at[j]], add=True)
            return 0

        lax.fori_loop(0, rows_per_tile // SUP, sup_body, 0)
        plsc.subcore_barrier()

        def wbody(i, _):
            r = s * STRIPE + i * WCH
            pltpu.sync_copy(dacc.at[pl.ds(r, WCH), :], wbuf.at[i % 2])
            pltpu.sync_copy(wbuf.at[i % 2], deg.at[c, pl.ds(r, WCH), :])
            return 0

        lax.fori_loop(0, STRIPE // WCH, wbody, 0)
        plsc.subcore_barrier()


@functools.cache
def _deg_kernel():
    return pl.kernel(
        _deg_body,
        out_type=(jax.ShapeDtypeStruct((NC, N, DEGW), jnp.float32),
                  jax.ShapeDtypeStruct((NC, N, DEGW), jnp.float32)),
        mesh=_mesh(),
        compiler_params=pltpu.CompilerParams(use_tc_tiling_on_sc=False),
        scratch_types=[
            pltpu.VMEM_SHARED((N, DEGW), jnp.float32),
            pltpu.VMEM((SUP, CHUNK), jnp.int32),
            pltpu.VMEM((CHUNK, DEGW), jnp.float32),
            pltpu.VMEM((2, WCH, DEGW), jnp.float32),
        ],
    )


def _deg2(dst2d_a, dst2d_b):
    return _deg_kernel()(dst2d_a, dst2d_b)


_BLK = 512


def _store_q(o_ref, z):
    for q in range(NQ):
        o_ref[q] = z[:, q * QW:(q + 1) * QW]


def _t0_body(x_ref, w_ref, b_ref, o_ref):
    z = jnp.maximum(
        jnp.dot(x_ref[...], w_ref[...], preferred_element_type=jnp.float32)
        + b_ref[...], 0.0)
    _store_q(o_ref, z)


def _t0(x_pad, w_pad, b):
    return pl.pallas_call(
        _t0_body,
        grid=(N // _BLK,),
        in_specs=[
            pl.BlockSpec((_BLK, 8), lambda i: (i, 0)),
            pl.BlockSpec((8, HID), lambda i: (0, 0)),
            pl.BlockSpec((1, HID), lambda i: (0, 0)),
        ],
        out_specs=pl.BlockSpec((NQ, _BLK, QW), lambda i: (0, i, 0)),
        out_shape=jax.ShapeDtypeStruct((NQ, N, QW), jnp.float32),
    )(x_pad, w_pad, b)


def _sage_body(agg_ref, x_ref, deg_ref, wl_ref, bl_ref, wr_ref, o_ref):
    d = deg_ref[0, :, 0:1] + deg_ref[1, :, 0:1]
    inv = 1.0 / jnp.maximum(d, 1.0)
    acc = bl_ref[...]
    for q in range(NQ):
        acc += jnp.dot(agg_ref[q] * inv, wl_ref[q],
                       preferred_element_type=jnp.float32)
        acc += jnp.dot(x_ref[q], wr_ref[q],
                       preferred_element_type=jnp.float32)
    _store_q(o_ref, jnp.maximum(acc, 0.0))


def _sage_update(agg, x4, deg, wl4, bl, wr4):
    return pl.pallas_call(
        _sage_body,
        grid=(N // _BLK,),
        in_specs=[
            pl.BlockSpec((NQ, _BLK, QW), lambda i: (0, i, 0)),
            pl.BlockSpec((NQ, _BLK, QW), lambda i: (0, i, 0)),
            pl.BlockSpec((NC, _BLK, DEGW), lambda i: (0, i, 0)),
            pl.BlockSpec((NQ, QW, HID), lambda i: (0, 0, 0)),
            pl.BlockSpec((1, HID), lambda i: (0, 0)),
            pl.BlockSpec((NQ, QW, HID), lambda i: (0, 0, 0)),
        ],
        out_specs=pl.BlockSpec((NQ, _BLK, QW), lambda i: (0, i, 0)),
        out_shape=jax.ShapeDtypeStruct((NQ, N, QW), jnp.float32),
    )(agg, x4, deg, wl4, bl, wr4)


_GBLK = 128


def _pool_body(h_ref, gf_ref, w1m_ref, w1x_ref, w1g_ref, b1_ref,
               w2_ref, b2_ref, o_ref):
    z = (jnp.dot(gf_ref[...], w1g_ref[...],
                 preferred_element_type=jnp.float32) + b1_ref[...])
    for q in range(NQ):
        ssum = h_ref[q, :, 0, :]
        smax = h_ref[q, :, 0, :]
        for j in range(1, ISL_PER_G):
            r = h_ref[q, :, j, :]
            ssum = ssum + r
            smax = jnp.maximum(smax, r)
        mp = ssum * (1.0 / ISL_PER_G)
        z += jnp.dot(mp, w1m_ref[q], preferred_element_type=jnp.float32)
        z += jnp.dot(smax, w1x_ref[q], preferred_element_type=jnp.float32)
    z = jnp.maximum(z, 0.0)
    z = jnp.dot(z, w2_ref[...], preferred_element_type=jnp.float32) + b2_ref[...]
    o_ref[...] = jnp.maximum(z, 0.0)


def _pool_mlp(h4d, gf_pad, w1m, w1x, w1g, b1, w2, b2):
    return pl.pallas_call(
        _pool_body,
        grid=(N_GI // _GBLK,),
        in_specs=[
            pl.BlockSpec((NQ, _GBLK, ISL_PER_G, QW), lambda i: (0, i, 0, 0)),
            pl.BlockSpec((_GBLK, 16), lambda i: (i, 0)),
            pl.BlockSpec((NQ, QW, HID), lambda i: (0, 0, 0)),
            pl.BlockSpec((NQ, QW, HID), lambda i: (0, 0, 0)),
            pl.BlockSpec((16, HID), lambda i: (0, 0)),
            pl.BlockSpec((1, HID), lambda i: (0, 0)),
            pl.BlockSpec((HID, HID), lambda i: (0, 0)),
            pl.BlockSpec((1, HID), lambda i: (0, 0)),
        ],
        out_specs=pl.BlockSpec((_GBLK, HID), lambda i: (i, 0)),
        out_shape=jax.ShapeDtypeStruct((N_GI, HID), jnp.float32),
    )(h4d, gf_pad, w1m, w1x, w1g, b1, w2, b2)


def _metax_body(t_ref, v_ref, isl_ref, seq_ref, o_ref):
    idx = t_ref[...] * NCELL + v_ref[...]
    lanes = lax.broadcasted_iota(jnp.int32, (_BLK, HID), 1)
    oh = (lanes == idx).astype(jnp.float32)
    z = (jnp.dot(oh[:, :NCELL], isl_ref[...],
                 preferred_element_type=jnp.float32)
         + jnp.dot(oh[:, NCELL:], seq_ref[...],
                   preferred_element_type=jnp.float32))
    _store_q(o_ref, z)


def _metax(t2d, v2d, island_emb, seq_emb):
    return pl.pallas_call(
        _metax_body,
        grid=(N // _BLK,),
        in_specs=[
            pl.BlockSpec((_BLK, 1), lambda i: (i, 0)),
            pl.BlockSpec((_BLK, 1), lambda i: (i, 0)),
            pl.BlockSpec((NCELL, HID), lambda i: (0, 0)),
            pl.BlockSpec((NCELL, HID), lambda i: (0, 0)),
        ],
        out_specs=pl.BlockSpec((NQ, _BLK, QW), lambda i: (0, i, 0)),
        out_shape=jax.ShapeDtypeStruct((NQ, N, QW), jnp.float32),
    )(t2d, v2d, island_emb, seq_emb)


_FBLK = 128


def _head_body(mh_ref, aidx_ref, anf_ref, wa_ref, ba_ref, wa2_ref, ba2_ref,
               wca_ref, wcw_ref, wcn_ref, bc_ref, wc2_ref, bc2_ref, o_ref):
    aidx = aidx_ref[...]
    a = jnp.maximum(
        jnp.dot(anf_ref[...], wa_ref[...], preferred_element_type=jnp.float32)
        + ba_ref[...], 0.0)
    a = jnp.maximum(
        jnp.dot(a, wa2_ref[...], preferred_element_type=jnp.float32)
        + ba2_ref[...], 0.0)
    z = (jnp.dot(a, wcn_ref[...], preferred_element_type=jnp.float32)
         + bc_ref[...])
    for q in range(NQ):
        ssum = mh_ref[q, :, 0, :]
        sel = jnp.where(aidx == 0, mh_ref[q, :, 0, :], 0.0)
        for j in range(1, NPW):
            r = mh_ref[q, :, j, :]
            ssum = ssum + r
            sel = jnp.where(aidx == j, r, sel)
        z += jnp.dot(sel, wca_ref[q], preferred_element_type=jnp.float32)
        z += jnp.dot(ssum * (1.0 / NPW), wcw_ref[q],
                     preferred_element_type=jnp.float32)
    z = jnp.maximum(z, 0.0)
    o_ref[...] = (jnp.dot(z, wc2_ref[...], preferred_element_type=jnp.float32)
                  + bc2_ref[...])


def _head(mh4d, aidx2d, anf, wa, ba, wa2, ba2, wca, wcw, wcn, bc, wc2, bc2):
    return pl.pallas_call(
        _head_body,
        grid=(B // _FBLK,),
        in_specs=[
            pl.BlockSpec((NQ, _FBLK, NPW, QW), lambda i: (0, i, 0, 0)),
            pl.BlockSpec((_FBLK, 1), lambda i: (i, 0)),
            pl.BlockSpec((_FBLK, 32), lambda i: (i, 0)),
            pl.BlockSpec((32, HID), lambda i: (0, 0)),
            pl.BlockSpec((1, HID), lambda i: (0, 0)),
            pl.BlockSpec((HID, HID), lambda i: (0, 0)),
            pl.BlockSpec((1, HID), lambda i: (0, 0)),
            pl.BlockSpec((NQ, QW, HID), lambda i: (0, 0, 0)),
            pl.BlockSpec((NQ, QW, HID), lambda i: (0, 0, 0)),
            pl.BlockSpec((HID, HID), lambda i: (0, 0)),
            pl.BlockSpec((1, HID), lambda i: (0, 0)),
            pl.BlockSpec((HID, 2), lambda i: (0, 0)),
            pl.BlockSpec((1, 2), lambda i: (0, 0)),
        ],
        out_specs=pl.BlockSpec((_FBLK, 2), lambda i: (i, 0)),
        out_shape=jax.ShapeDtypeStruct((B, 2), jnp.float32),
    )(mh4d, aidx2d, anf, wa, ba, wa2, ba2, wca, wcw, wcn, bc, wc2, bc2)


def kernel(anchor_num_feat, island_x, island_graph_feat, params,
           island_edge_index, island_batch_vec, meta_x_type, meta_x_val,
           meta_edge_index, meta_ptr, meta_batch_vec, anchor_node_idx):
    del island_batch_vec, meta_ptr, meta_batch_vec

    isrc = island_edge_index[0].reshape(E // CHUNK, CHUNK)
    idst = island_edge_index[1].reshape(E // CHUNK, CHUNK)
    msrc = meta_edge_index[0].reshape(E // CHUNK, CHUNK)
    mdst = meta_edge_index[1].reshape(E // CHUNK, CHUNK)

    w_in, b_in = params["in_proj"]
    x_pad = jnp.pad(island_x, ((0, 0), (0, 2)))
    w_pad = jnp.pad(w_in, ((0, 2), (0, 0)))

    def q4(w):
        return w.reshape(NQ, QW, -1)

    h = _t0(x_pad, w_pad, b_in.reshape(1, HID))
    deg_i, deg_m = _deg2(idst, mdst)
    for (wl, bl, wr) in params["isl_sage"]:
        agg = _seg_sum(h.reshape(NQ * N, QW), isrc, idst)
        h = _sage_update(agg, h, deg_i, q4(wl), bl.reshape(1, HID), q4(wr))

    w1, b1 = params["out1"]
    w2, b2 = params["out2"]
    gf_pad = jnp.pad(island_graph_feat, ((0, 0), (0, 16 - 9)))
    w1g_pad = jnp.pad(w1[2 * HID:], ((0, 16 - 9), (0, 0)))
    island_emb = _pool_mlp(
        h.reshape(NQ, N_GI, ISL_PER_G, QW), gf_pad,
        q4(w1[:HID]), q4(w1[HID:2 * HID]), w1g_pad, b1.reshape(1, HID),
        w2, b2.reshape(1, HID))

    mh = _metax(meta_x_type.reshape(N, 1), meta_x_val.reshape(N, 1),
                island_emb[:NCELL], params["seq_emb"])
    for (wl, bl, wr) in params["meta_sage"]:
        agg = _seg_sum(mh.reshape(NQ * N, QW), msrc, mdst)
        mh = _sage_update(agg, mh, deg_m, q4(wl), bl.reshape(1, HID), q4(wr))

    wa, ba = params["anc1"]
    wa2, ba2 = params["anc2"]
    wc, bc = params["cls1"]
    wc2, bc2 = params["cls2"]
    return _head(
        mh.reshape(NQ, B, NPW, QW), anchor_node_idx.reshape(B, 1),
        anchor_num_feat,
        wa, ba.reshape(1, HID), wa2, ba2.reshape(1, HID),
        q4(wc[:HID]), q4(wc[HID:2 * HID]), wc[2 * HID:], bc.reshape(1, HID),
        wc2, bc2.reshape(1, 2))


# speedup vs baseline: 1.3856x; 1.3856x over previous
import functools

import jax
import jax.numpy as jnp
from jax import lax
from jax.experimental import pallas as pl
from jax.experimental.pallas import tpu as pltpu
from jax.experimental.pallas import tpu_sc as plsc

HID = 128
N = 49152
E = 786432
N_GI = 2048
B = 1024
NPW = 48
ISL_PER_G = 24
NCELL = 64
QW = 32
NQ = 4
NC = 2
NS = 16
CHUNK = 128
SUP = 8
ROWS_PER_TILE = E // CHUNK // NS
N_SUP = ROWS_PER_TILE // SUP
STRIPE = N // NS
WCH = 128
DEGW = 16


@functools.cache
def _mesh():
    return plsc.VectorSubcoreMesh(core_axis_name="c", subcore_axis_name="s",
                                  num_cores=NC, num_subcores=NS)


def _fill2d16(ref, rows, width, value):
    v = jnp.full((16,), value, jnp.float32)
    nw = width // 16

    def body(k, _):
        ref[k // nw, pl.ds((k % nw) * 16, 16)] = v
        return 0

    lax.fori_loop(0, rows * nw, body, 0)


def _seg_sum_body(xflat, src2d, dst2d, agg, acc, srcv, dstv, gidx, rows,
                  wbuf, gs0, gs1, gs2, gs3, ss0, ss1, ss2, ss3, is0, is1,
                  zs0, ws0, ws1):
    c = lax.axis_index("c")
    s = lax.axis_index("s")
    gsems = (gs0, gs1, gs2, gs3)
    ssems = (ss0, ss1, ss2, ss3)
    isems = (is0, is1)
    wsems = (ws0, ws1)
    base0 = s * ROWS_PER_TILE
    nz = STRIPE // WCH

    _fill2d16(wbuf.at[0], WCH, QW, 0.0)

    for qq in range(2):
        q = c * 2 + qq

        def zfire(i, _):
            pltpu.async_copy(wbuf.at[0],
                             acc.at[pl.ds(s * STRIPE + i * WCH, WCH), :],
                             zs0)
            return 0

        lax.fori_loop(0, nz, zfire, 0)

        def zdrain(i, _):
            pltpu.make_async_copy(
                wbuf.at[0], acc.at[pl.ds(s * STRIPE, WCH), :], zs0).wait()
            return 0

        lax.fori_loop(0, nz, zdrain, 0)
        plsc.subcore_barrier()

        def load_idx(sup, p, sync):
            rr = base0 + sup * SUP
            if sync:
                pltpu.sync_copy(src2d.at[pl.ds(rr, SUP)], srcv.at[p])
                pltpu.sync_copy(dst2d.at[pl.ds(rr, SUP)], dstv.at[p])
            else:
                pltpu.async_copy(src2d.at[pl.ds(rr, SUP)], srcv.at[p],
                                 isems[p])
                pltpu.async_copy(dst2d.at[pl.ds(rr, SUP)], dstv.at[p],
                                 isems[p])

        def wait_idx(p):
            pltpu.make_async_copy(src2d.at[pl.ds(0, SUP)], srcv.at[p],
                                  isems[p]).wait()
            pltpu.make_async_copy(dst2d.at[pl.ds(0, SUP)], dstv.at[p],
                                  isems[p]).wait()

        nv = CHUNK // 16

        def compute_gidx(p):
            def idx_body(k, _):
                j = k // nv
                o = (k % nv) * 16
                gidx[p, j, pl.ds(o, 16)] = srcv[p, j, pl.ds(o, 16)] * 4 + q
                return 0

            lax.fori_loop(0, SUP * nv, idx_body, 0)

        def fire_gather(p, j, b):
            pltpu.async_copy(xflat.at[gidx.at[p, j]], rows.at[b], gsems[b])

        def fire_scatter(p, j, b):
            pltpu.async_copy(rows.at[b], acc.at[dstv.at[p, j]],
                             ssems[b], add=True)

        def gwait(b):
            pltpu.make_async_copy(xflat.at[gidx.at[0, 0]], rows.at[b],
                                  gsems[b]).wait()

        def swait(b):
            pltpu.make_async_copy(rows.at[b], acc.at[dstv.at[0, 0]],
                                  ssems[b]).wait()

        def rolling_sup(sup, p, first, last):
            def slot(j):
                b = j % 4
                bp2 = (j + 2) % 4
                gwait(b)
                fire_scatter(p, j, b)
                if not (last and j >= 6):
                    if not (first and j < 2):
                        swait(bp2)
                    if j < 6:
                        fire_gather(p, j + 2, bp2)
                    else:
                        fire_gather(p ^ 1, j - 6, bp2)

            slot(0)
            slot(1)
            if not last:
                load_idx(sup + 1, p ^ 1, sync=False)
            for j in range(2, 6):
                slot(j)
            if not last:
                wait_idx(p ^ 1)
                compute_gidx(p ^ 1)
            slot(6)
            slot(7)

        load_idx(0, 0, sync=True)
        compute_gidx(0)
        fire_gather(0, 0, 0)
        fire_gather(0, 1, 1)
        rolling_sup(0, 0, True, False)

        def pair_body(t, _):
            sup = 1 + 2 * t
            rolling_sup(sup, 1, False, False)
            rolling_sup(sup + 1, 0, False, False)
            return 0

        lax.fori_loop(0, (N_SUP - 2) // 2, pair_body, 0)

        rolling_sup(N_SUP - 1, 1, False, True)
        for b in range(4):
            swait(b)
        plsc.subcore_barrier()

        def wchunk(i, p, wait):
            r = s * STRIPE + i * WCH
            if wait:
                pltpu.make_async_copy(
                    wbuf.at[p], agg.at[q, pl.ds(s * STRIPE, WCH), :],
                    wsems[p]).wait()
            pltpu.sync_copy(acc.at[pl.ds(r, WCH), :], wbuf.at[p])
            pltpu.async_copy(wbuf.at[p], agg.at[q, pl.ds(r, WCH), :],
                             wsems[p])

        wchunk(0, 0, False)
        wchunk(1, 1, False)

        def wbody(t, _):
            wchunk(2 + 2 * t, 0, True)
            wchunk(3 + 2 * t, 1, True)
            return 0

        lax.fori_loop(0, (nz - 2) // 2, wbody, 0)
        for p in range(2):
            pltpu.make_async_copy(
                wbuf.at[p], agg.at[q, pl.ds(s * STRIPE, WCH), :],
                wsems[p]).wait()
        if qq == 0:
            _fill2d16(wbuf.at[0], WCH, QW, 0.0)
        plsc.subcore_barrier()


@functools.cache
def _seg_sum_kernel():
    return pl.kernel(
        _seg_sum_body,
        out_type=jax.ShapeDtypeStruct((NQ, N, QW), jnp.float32),
        mesh=_mesh(),
        compiler_params=pltpu.CompilerParams(use_tc_tiling_on_sc=False),
        scratch_types=[
            pltpu.VMEM_SHARED((N, QW), jnp.float32),
            pltpu.VMEM((2, SUP, CHUNK), jnp.int32),
            pltpu.VMEM((2, SUP, CHUNK), jnp.int32),
            pltpu.VMEM((2, SUP, CHUNK), jnp.int32),
            pltpu.VMEM((4, CHUNK, QW), jnp.float32),
            pltpu.VMEM((2, WCH, QW), jnp.float32),
        ] + [pltpu.SemaphoreType.DMA] * 13,
    )


def _seg_sum(x2d, src2d, dst2d):
    return _seg_sum_kernel()(x2d, src2d, dst2d)


def _deg_body(dst2d_a, dst2d_b, deg_a, deg_b, dacc, dstv, onesv, wbuf):
    c = lax.axis_index("c")
    s = lax.axis_index("s")

    _fill2d16(onesv, CHUNK, DEGW, 1.0)

    rows_per_tile = E // CHUNK // (NC * NS)

    for dst2d, deg in ((dst2d_a, deg_a), (dst2d_b, deg_b)):
        _fill2d16(wbuf.at[0], WCH, DEGW, 0.0)

        def zbody(i, _):
            pltpu.sync_copy(wbuf.at[0],
                            dacc.at[pl.ds(s * STRIPE + i * WCH, WCH), :])
            return 0

        lax.fori_loop(0, STRIPE // WCH, zbody, 0)
        plsc.subcore_barrier()

        def sup_body(t, _):
            base = (c * NS + s) * rows_per_tile + t * SUP
            pltpu.sync_copy(dst2d.at[pl.ds(base, SUP)], dstv)
            for j in range(SUP):
                pltpu.sync_copy(onesv, dacc.at[dstv.at[j]], add=True)
            return 0

        lax.fori_loop(0, rows_per_tile // SUP, sup_body, 0)
        plsc.subcore_barrier()

        def wbody(i, _):
            r = s * STRIPE + i * WCH
            pltpu.sync_copy(dacc.at[pl.ds(r, WCH), :], wbuf.at[i % 2])
            pltpu.sync_copy(wbuf.at[i % 2], deg.at[c, pl.ds(r, WCH), :])
            return 0

        lax.fori_loop(0, STRIPE // WCH, wbody, 0)
        plsc.subcore_barrier()


@functools.cache
def _deg_kernel():
    return pl.kernel(
        _deg_body,
        out_type=(jax.ShapeDtypeStruct((NC, N, DEGW), jnp.float32),
                  jax.ShapeDtypeStruct((NC, N, DEGW), jnp.float32)),
        mesh=_mesh(),
        compiler_params=pltpu.CompilerParams(use_tc_tiling_on_sc=False),
        scratch_types=[
            pltpu.VMEM_SHARED((N, DEGW), jnp.float32),
            pltpu.VMEM((SUP, CHUNK), jnp.int32),
            pltpu.VMEM((CHUNK, DEGW), jnp.float32),
            pltpu.VMEM((2, WCH, DEGW), jnp.float32),
        ],
    )


def _deg2(dst2d_a, dst2d_b):
    return _deg_kernel()(dst2d_a, dst2d_b)


_BLK = 2048


def _t0_body(x_ref, w_ref, b_ref, o_ref):
    o_ref[...] = jnp.maximum(
        jnp.dot(x_ref[...], w_ref[...], preferred_element_type=jnp.float32)
        + b_ref[...], 0.0)


def _t0(x_pad, w_pad, b):
    return pl.pallas_call(
        _t0_body,
        grid=(N // _BLK,),
        in_specs=[
            pl.BlockSpec((_BLK, 8), lambda i: (i, 0)),
            pl.BlockSpec((8, HID), lambda i: (0, 0)),
            pl.BlockSpec((1, HID), lambda i: (0, 0)),
        ],
        out_specs=pl.BlockSpec((_BLK, HID), lambda i: (i, 0)),
        out_shape=jax.ShapeDtypeStruct((N, HID), jnp.float32),
    )(x_pad, w_pad, b)


def _sage_body(agg_ref, x_ref, deg_ref, wl_ref, bl_ref, wr_ref, o_ref):
    d = deg_ref[0, :, 0:1] + deg_ref[1, :, 0:1]
    inv = 1.0 / jnp.maximum(d, 1.0)
    acc = jnp.dot(x_ref[...], wr_ref[...], preferred_element_type=jnp.float32)
    for q in range(NQ):
        acc += jnp.dot(agg_ref[q] * inv, wl_ref[q],
                       preferred_element_type=jnp.float32)
    o_ref[...] = jnp.maximum(acc + bl_ref[...], 0.0)


def _sage_update(agg, x, deg, wl4, bl, wr):
    return pl.pallas_call(
        _sage_body,
        grid=(N // _BLK,),
        in_specs=[
            pl.BlockSpec((NQ, _BLK, QW), lambda i: (0, i, 0)),
            pl.BlockSpec((_BLK, HID), lambda i: (i, 0)),
            pl.BlockSpec((NC, _BLK, DEGW), lambda i: (0, i, 0)),
            pl.BlockSpec((NQ, QW, HID), lambda i: (0, 0, 0)),
            pl.BlockSpec((1, HID), lambda i: (0, 0)),
            pl.BlockSpec((HID, HID), lambda i: (0, 0)),
        ],
        out_specs=pl.BlockSpec((_BLK, HID), lambda i: (i, 0)),
        out_shape=jax.ShapeDtypeStruct((N, HID), jnp.float32),
    )(agg, x, deg, wl4, bl, wr)


_GBLK = 128


def _pool_body(h_ref, gf_ref, w1m_ref, w1x_ref, w1g_ref, b1_ref,
               w2_ref, b2_ref, o_ref):
    ssum = h_ref[:, 0, :]
    smax = h_ref[:, 0, :]
    for j in range(1, ISL_PER_G):
        r = h_ref[:, j, :]
        ssum = ssum + r
        smax = jnp.maximum(smax, r)
    mp = ssum * (1.0 / ISL_PER_G)
    z = (jnp.dot(mp, w1m_ref[...], preferred_element_type=jnp.float32)
         + jnp.dot(smax, w1x_ref[...], preferred_element_type=jnp.float32)
         + jnp.dot(gf_ref[...], w1g_ref[...],
                   preferred_element_type=jnp.float32)
         + b1_ref[...])
    z = jnp.maximum(z, 0.0)
    z = jnp.dot(z, w2_ref[...], preferred_element_type=jnp.float32) + b2_ref[...]
    o_ref[...] = jnp.maximum(z, 0.0)


def _pool_mlp(h3d, gf_pad, w1m, w1x, w1g, b1, w2, b2):
    return pl.pallas_call(
        _pool_body,
        grid=(N_GI // _GBLK,),
        in_specs=[
            pl.BlockSpec((_GBLK, ISL_PER_G, HID), lambda i: (i, 0, 0)),
            pl.BlockSpec((_GBLK, 16), lambda i: (i, 0)),
            pl.BlockSpec((HID, HID), lambda i: (0, 0)),
            pl.BlockSpec((HID, HID), lambda i: (0, 0)),
            pl.BlockSpec((16, HID), lambda i: (0, 0)),
            pl.BlockSpec((1, HID), lambda i: (0, 0)),
            pl.BlockSpec((HID, HID), lambda i: (0, 0)),
            pl.BlockSpec((1, HID), lambda i: (0, 0)),
        ],
        out_specs=pl.BlockSpec((_GBLK, HID), lambda i: (i, 0)),
        out_shape=jax.ShapeDtypeStruct((N_GI, HID), jnp.float32),
    )(h3d, gf_pad, w1m, w1x, w1g, b1, w2, b2)


def _metax_body(t_ref, v_ref, isl_ref, seq_ref, o_ref):
    idx = t_ref[...] * NCELL + v_ref[...]
    lanes = lax.broadcasted_iota(jnp.int32, (_BLK, HID), 1)
    oh = (lanes == idx).astype(jnp.float32)
    o_ref[...] = (
        jnp.dot(oh[:, :NCELL], isl_ref[...],
                preferred_element_type=jnp.float32)
        + jnp.dot(oh[:, NCELL:], seq_ref[...],
                  preferred_element_type=jnp.float32))


def _metax(t2d, v2d, island_emb, seq_emb):
    return pl.pallas_call(
        _metax_body,
        grid=(N // _BLK,),
        in_specs=[
            pl.BlockSpec((_BLK, 1), lambda i: (i, 0)),
            pl.BlockSpec((_BLK, 1), lambda i: (i, 0)),
            pl.BlockSpec((NCELL, HID), lambda i: (0, 0)),
            pl.BlockSpec((NCELL, HID), lambda i: (0, 0)),
        ],
        out_specs=pl.BlockSpec((_BLK, HID), lambda i: (i, 0)),
        out_shape=jax.ShapeDtypeStruct((N, HID), jnp.float32),
    )(t2d, v2d, island_emb, seq_emb)


_FBLK = 128


def _head_body(mh_ref, aidx_ref, anf_ref, wa_ref, ba_ref, wa2_ref, ba2_ref,
               wca_ref, wcw_ref, wcn_ref, bc_ref, wc2_ref, bc2_ref, o_ref):
    aidx = aidx_ref[...]
    ssum = mh_ref[:, 0, :]
    sel = jnp.where(aidx == 0, mh_ref[:, 0, :], 0.0)
    for j in range(1, NPW):
        r = mh_ref[:, j, :]
        ssum = ssum + r
        sel = jnp.where(aidx == j, r, sel)
    wemb = ssum * (1.0 / NPW)
    a = jnp.maximum(
        jnp.dot(anf_ref[...], wa_ref[...], preferred_element_type=jnp.float32)
        + ba_ref[...], 0.0)
    a = jnp.maximum(
        jnp.dot(a, wa2_ref[...], preferred_element_type=jnp.float32)
        + ba2_ref[...], 0.0)
    z = (jnp.dot(sel, wca_ref[...], preferred_element_type=jnp.float32)
         + jnp.dot(wemb, wcw_ref[...], preferred_element_type=jnp.float32)
         + jnp.dot(a, wcn_ref[...], preferred_element_type=jnp.float32)
         + bc_ref[...])
    z = jnp.maximum(z, 0.0)
    o_ref[...] = (jnp.dot(z, wc2_ref[...], preferred_element_type=jnp.float32)
                  + bc2_ref[...])


def _head(mh3d, aidx2d, anf, wa, ba, wa2, ba2, wca, wcw, wcn, bc, wc2, bc2):
    return pl.pallas_call(
        _head_body,
        grid=(B // _FBLK,),
        in_specs=[
            pl.BlockSpec((_FBLK, NPW, HID), lambda i: (i, 0, 0)),
            pl.BlockSpec((_FBLK, 1), lambda i: (i, 0)),
            pl.BlockSpec((_FBLK, 32), lambda i: (i, 0)),
            pl.BlockSpec((32, HID), lambda i: (0, 0)),
            pl.BlockSpec((1, HID), lambda i: (0, 0)),
            pl.BlockSpec((HID, HID), lambda i: (0, 0)),
            pl.BlockSpec((1, HID), lambda i: (0, 0)),
            pl.BlockSpec((HID, HID), lambda i: (0, 0)),
            pl.BlockSpec((HID, HID), lambda i: (0, 0)),
            pl.BlockSpec((HID, HID), lambda i: (0, 0)),
            pl.BlockSpec((1, HID), lambda i: (0, 0)),
            pl.BlockSpec((HID, 2), lambda i: (0, 0)),
            pl.BlockSpec((1, 2), lambda i: (0, 0)),
        ],
        out_specs=pl.BlockSpec((_FBLK, 2), lambda i: (i, 0)),
        out_shape=jax.ShapeDtypeStruct((B, 2), jnp.float32),
    )(mh3d, aidx2d, anf, wa, ba, wa2, ba2, wca, wcw, wcn, bc, wc2, bc2)


def kernel(anchor_num_feat, island_x, island_graph_feat, params,
           island_edge_index, island_batch_vec, meta_x_type, meta_x_val,
           meta_edge_index, meta_ptr, meta_batch_vec, anchor_node_idx):
    del island_batch_vec, meta_ptr, meta_batch_vec

    isrc = island_edge_index[0].reshape(E // CHUNK, CHUNK)
    idst = island_edge_index[1].reshape(E // CHUNK, CHUNK)
    msrc = meta_edge_index[0].reshape(E // CHUNK, CHUNK)
    mdst = meta_edge_index[1].reshape(E // CHUNK, CHUNK)

    w_in, b_in = params["in_proj"]
    x_pad = jnp.pad(island_x, ((0, 0), (0, 2)))
    w_pad = jnp.pad(w_in, ((0, 2), (0, 0)))

    def q4(w):
        return w.reshape(NQ, QW, -1)

    h = _t0(x_pad, w_pad, b_in.reshape(1, HID))
    deg_i, deg_m = _deg2(idst, mdst)
    for (wl, bl, wr) in params["isl_sage"]:
        agg = _seg_sum(h.reshape(NQ * N, QW), isrc, idst)
        h = _sage_update(agg, h, deg_i, q4(wl), bl.reshape(1, HID), wr)

    w1, b1 = params["out1"]
    w2, b2 = params["out2"]
    gf_pad = jnp.pad(island_graph_feat, ((0, 0), (0, 16 - 9)))
    w1g_pad = jnp.pad(w1[2 * HID:], ((0, 16 - 9), (0, 0)))
    island_emb = _pool_mlp(
        h.reshape(N_GI, ISL_PER_G, HID), gf_pad,
        w1[:HID], w1[HID:2 * HID], w1g_pad, b1.reshape(1, HID),
        w2, b2.reshape(1, HID))

    mh = _metax(meta_x_type.reshape(N, 1), meta_x_val.reshape(N, 1),
                island_emb[:NCELL], params["seq_emb"])
    for (wl, bl, wr) in params["meta_sage"]:
        agg = _seg_sum(mh.reshape(NQ * N, QW), msrc, mdst)
        mh = _sage_update(agg, mh, deg_m, q4(wl), bl.reshape(1, HID), wr)

    wa, ba = params["anc1"]
    wa2, ba2 = params["anc2"]
    wc, bc = params["cls1"]
    wc2, bc2 = params["cls2"]
    return _head(
        mh.reshape(B, NPW, HID), anchor_node_idx.reshape(B, 1),
        anchor_num_feat,
        wa, ba.reshape(1, HID), wa2, ba2.reshape(1, HID),
        wc[:HID], wc[HID:2 * HID], wc[2 * HID:], bc.reshape(1, HID),
        wc2, bc2.reshape(1, 2))
